# baseline (speedup 1.0000x reference)
import jax
import jax.numpy as jnp
from jax.experimental import pallas as pl

_B, _EMB = 1024, 768
_M, _K, _D = 96, 256, 8
_BB = 128
_CD = 6 * 2 * _D


def _body(lhs_ref, rhs_ref, cbt_ref, out_ref):
    lhs = lhs_ref[...]
    rhs = rhs_ref[...]
    cbt = cbt_ref[...]
    adj = jax.lax.dot_general(
        lhs, rhs, (((2,), (1,)), ((0,), (0,))),
        preferred_element_type=jnp.float32,
        precision=jax.lax.Precision.DEFAULT)
    amax = jnp.max(adj, axis=1, keepdims=True)
    oh = (adj == amax).astype(jnp.float32)
    out_ref[...] = jax.lax.dot_general(
        cbt, oh, (((2,), (1,)), ((0,), (0,))),
        preferred_element_type=jnp.float32,
        precision=jax.lax.Precision.DEFAULT)


def _limbs(x):
    h = x.astype(jnp.bfloat16)
    m = (x - h.astype(jnp.float32)).astype(jnp.bfloat16)
    l = (x - h.astype(jnp.float32) - m.astype(jnp.float32)).astype(jnp.bfloat16)
    return h, m, l


def kernel(vecs, codebook):
    vt = vecs.reshape(_B, _M, _D).transpose(1, 2, 0)
    cbx = jnp.concatenate([codebook, -0.5 * codebook * codebook],
                          axis=2)
    ones = jnp.ones((_M, _D, _B), dtype=jnp.float32)
    vx = jnp.concatenate([vt, ones], axis=1)
    ch, cm, cl = _limbs(cbx)
    vh, vm, vl = _limbs(vx)
    lhs = jnp.concatenate([ch, ch, cm, ch, cl, cm], axis=2)
    rhs = jnp.concatenate([vh, vm, vh, vl, vh, vm], axis=1)
    cbt = codebook.transpose(0, 2, 1)
    q = pl.pallas_call(
        _body,
        grid=(_B // _BB,),
        in_specs=[
            pl.BlockSpec((_M, _K, _CD), lambda i: (0, 0, 0)),
            pl.BlockSpec((_M, _CD, _BB), lambda i: (0, 0, i)),
            pl.BlockSpec((_M, _D, _K), lambda i: (0, 0, 0)),
        ],
        out_specs=pl.BlockSpec((_M, _D, _BB), lambda i: (0, 0, i)),
        out_shape=jax.ShapeDtypeStruct((_M, _D, _B), jnp.float32),
    )(lhs, rhs, cbt)
    return q.transpose(2, 0, 1).reshape(_B, _EMB)


# speedup vs baseline: 1.0021x; 1.0021x over previous
import jax
import jax.numpy as jnp
from jax.experimental import pallas as pl

_B, _EMB = 1024, 768
_M, _K, _D = 96, 256, 8
_BB = 128
_CD = 6 * 2 * _D


def _body(lhs_ref, rhs_ref, cbt_ref, out_ref):
    lhs = lhs_ref[...]
    rhs = rhs_ref[...]
    cbt = cbt_ref[...]
    adj = jax.lax.dot_general(
        lhs, rhs, (((2,), (1,)), ((0,), (0,))),
        preferred_element_type=jnp.float32,
        precision=jax.lax.Precision.DEFAULT)
    amax = jnp.max(adj, axis=1, keepdims=True)
    oh = (adj == amax).astype(jnp.float32)
    out_ref[...] = jax.lax.dot_general(
        cbt, oh, (((2,), (1,)), ((0,), (0,))),
        preferred_element_type=jnp.float32,
        precision=jax.lax.Precision.DEFAULT)


def _trunc16(x):
    return jax.lax.bitcast_convert_type(
        jax.lax.bitcast_convert_type(x, jnp.uint32) & jnp.uint32(0xFFFF0000),
        jnp.float32)


def _limbs(x):
    h = _trunc16(x)
    r = x - h
    m = _trunc16(r)
    l = r - m
    return (h.astype(jnp.bfloat16), m.astype(jnp.bfloat16),
            l.astype(jnp.bfloat16))


def kernel(vecs, codebook):
    vt = vecs.reshape(_B, _M, _D).transpose(1, 2, 0)
    cbx = jnp.concatenate([codebook, -0.5 * codebook * codebook],
                          axis=2)
    ones = jnp.ones((_M, _D, _B), dtype=jnp.float32)
    vx = jnp.concatenate([vt, ones], axis=1)
    ch, cm, cl = _limbs(cbx)
    vh, vm, vl = _limbs(vx)
    lhs = jnp.concatenate([ch, ch, cm, ch, cl, cm], axis=2)
    rhs = jnp.concatenate([vh, vm, vh, vl, vh, vm], axis=1)
    cbt = codebook.transpose(0, 2, 1)
    q = pl.pallas_call(
        _body,
        grid=(_B // _BB,),
        in_specs=[
            pl.BlockSpec((_M, _K, _CD), lambda i: (0, 0, 0)),
            pl.BlockSpec((_M, _CD, _BB), lambda i: (0, 0, i)),
            pl.BlockSpec((_M, _D, _K), lambda i: (0, 0, 0)),
        ],
        out_specs=pl.BlockSpec((_M, _D, _BB), lambda i: (0, 0, i)),
        out_shape=jax.ShapeDtypeStruct((_M, _D, _B), jnp.float32),
    )(lhs, rhs, cbt)
    return q.transpose(2, 0, 1).reshape(_B, _EMB)


# speedup vs baseline: 1.0607x; 1.0585x over previous
import jax
import jax.numpy as jnp
from jax.experimental import pallas as pl

_B, _EMB = 1024, 768
_M, _K, _D = 96, 256, 8
_BB = 128
_D2 = 2 * _D
_CD = 6 * _D2


def _trunc16(x):
    return jax.lax.bitcast_convert_type(
        jax.lax.bitcast_convert_type(x, jnp.uint32) & jnp.uint32(0xFFFF0000),
        jnp.float32)


def _limbs(x):
    h = _trunc16(x)
    r = x - h
    m = _trunc16(r)
    l = r - m
    return (h.astype(jnp.bfloat16), m.astype(jnp.bfloat16),
            l.astype(jnp.bfloat16))


def _body(vx_ref, lhs_ref, cbt_ref, out_ref):
    vx = vx_ref[...]
    lhs = lhs_ref[...]
    cbt = cbt_ref[...]
    vh, vm, vl = _limbs(vx)
    rhs = jnp.concatenate([vh, vm, vh, vl, vh, vm], axis=1)
    adj = jax.lax.dot_general(
        lhs, rhs, (((2,), (1,)), ((0,), (0,))),
        preferred_element_type=jnp.float32,
        precision=jax.lax.Precision.DEFAULT)
    amax = jnp.max(adj, axis=1, keepdims=True)
    oh = (adj == amax).astype(jnp.float32)
    out_ref[...] = jax.lax.dot_general(
        cbt, oh, (((2,), (1,)), ((0,), (0,))),
        preferred_element_type=jnp.float32,
        precision=jax.lax.Precision.DEFAULT)


def kernel(vecs, codebook):
    vt = vecs.reshape(_B, _M, _D).transpose(1, 2, 0)
    ones = jnp.ones((_M, _D, _B), dtype=jnp.float32)
    vx = jnp.concatenate([vt, ones], axis=1)
    cbx = jnp.concatenate([codebook, -0.5 * codebook * codebook],
                          axis=2)
    ch, cm, cl = _limbs(cbx)
    lhs = jnp.concatenate([ch, ch, cm, ch, cl, cm], axis=2)
    cbt = codebook.transpose(0, 2, 1)
    q = pl.pallas_call(
        _body,
        grid=(_B // _BB,),
        in_specs=[
            pl.BlockSpec((_M, _D2, _BB), lambda i: (0, 0, i)),
            pl.BlockSpec((_M, _K, _CD), lambda i: (0, 0, 0)),
            pl.BlockSpec((_M, _D, _K), lambda i: (0, 0, 0)),
        ],
        out_specs=pl.BlockSpec((_M, _D, _BB), lambda i: (0, 0, i)),
        out_shape=jax.ShapeDtypeStruct((_M, _D, _B), jnp.float32),
    )(vx, lhs, cbt)
    return q.transpose(2, 0, 1).reshape(_B, _EMB)
